# Initial kernel scaffold; baseline (speedup 1.0000x reference)
import functools

import jax
import jax.numpy as jnp
from jax import lax
from jax.experimental import pallas as pl
from jax.experimental.pallas import tpu as pltpu
from jax.experimental.pallas import tpu_sc as plsc

_INPUT_SIZE = 41024
_L1 = 256
_BATCH = 16384
_N_ACTIVE = 32

_NC = 2
_NS = 16
_NW = _NC * _NS

_TOTAL_BAGS = 2 * _BATCH
_BAGS_PER_W = _TOTAL_BAGS // _NW
_C = 4
_NCHUNK = _BAGS_PER_W // _C
_ROWS_PER_CHUNK = _C * _N_ACTIVE


def _sc_bag_sum_body(idx_hbm, bias_hbm, emb_hbm, out_hbm,
                     idx_v, rows_v, acc_v, bias_v, sem0, sem1):
    wid = lax.axis_index("s") * _NC + lax.axis_index("c")
    base_bag = wid * _BAGS_PER_W

    pltpu.sync_copy(idx_hbm.at[pl.ds(base_bag * _N_ACTIVE, _BAGS_PER_W * _N_ACTIVE)],
                    idx_v)
    pltpu.sync_copy(bias_hbm, bias_v)

    sems = (sem0, sem1)

    def issue_gather(c, b):
        off = c * _ROWS_PER_CHUNK
        pltpu.async_copy(emb_hbm.at[idx_v.at[pl.ds(off, _ROWS_PER_CHUNK)]],
                         rows_v.at[b], sems[b])

    issue_gather(0, 0)
    issue_gather(1, 1)

    def outer(i, carry):
        for b in range(2):
            c = 2 * i + b
            pltpu.make_async_copy(emb_hbm.at[pl.ds(0, _ROWS_PER_CHUNK)],
                                  rows_v.at[b], sems[b]).wait()
            for j in range(_C):
                def row_add(r, acc, _j=j, _b=b):
                    return tuple(
                        acc[k] + rows_v[_b, _j * _N_ACTIVE + r, pl.ds(k * 16, 16)]
                        for k in range(_L1 // 16))
                acc0 = tuple(bias_v[pl.ds(k * 16, 16)] for k in range(_L1 // 16))
                acc = lax.fori_loop(0, _N_ACTIVE, row_add, acc0)
                for k in range(_L1 // 16):
                    v = jnp.minimum(jnp.maximum(acc[k], 0.0), 1.0)
                    acc_v[j, pl.ds(k * 16, 16)] = v
            pltpu.sync_copy(acc_v, out_hbm.at[pl.ds(base_bag + c * _C, _C)])
            @pl.when(c + 2 < _NCHUNK)
            def _():
                issue_gather(c + 2, b)
        return carry

    lax.fori_loop(0, _NCHUNK // 2, outer, 0)


_sc_bag_sum = functools.partial(
    pl.kernel,
    out_type=jax.ShapeDtypeStruct((_TOTAL_BAGS, _L1), jnp.float32),
    mesh=plsc.VectorSubcoreMesh(core_axis_name="c", subcore_axis_name="s"),
    scratch_types=[
        pltpu.VMEM((_BAGS_PER_W * _N_ACTIVE,), jnp.int32),
        pltpu.VMEM((2, _ROWS_PER_CHUNK, _L1), jnp.float32),
        pltpu.VMEM((_C, _L1), jnp.float32),
        pltpu.VMEM((_L1,), jnp.float32),
        pltpu.SemaphoreType.DMA,
        pltpu.SemaphoreType.DMA,
    ],
)(_sc_bag_sum_body)


def _mlp_body(ys_ref, yn_ref, w1s_ref, w1n_ref, b1_ref, w2_ref, b2_ref,
              wo_ref, bo_ref, out_ref):
    dn = (((1,), (1,)), ((), ()))
    h = (lax.dot_general(ys_ref[...], w1s_ref[...], dn,
                         preferred_element_type=jnp.float32)
         + lax.dot_general(yn_ref[...], w1n_ref[...], dn,
                           preferred_element_type=jnp.float32)
         + b1_ref[...])
    h = jnp.clip(h, 0.0, 1.0)
    h = lax.dot_general(h, w2_ref[...], dn,
                        preferred_element_type=jnp.float32) + b2_ref[...]
    h = jnp.clip(h, 0.0, 1.0)
    o = lax.dot_general(h, wo_ref[...], dn,
                        preferred_element_type=jnp.float32) + bo_ref[...]
    out_ref[...] = jax.nn.sigmoid(o)


_BB = 2048


def _mlp(y, w1s, w1n, b1, w2, b2, wo, bo):
    grid = (_BATCH // _BB,)
    return pl.pallas_call(
        _mlp_body,
        grid=grid,
        in_specs=[
            pl.BlockSpec((_BB, _L1), lambda i: (i, 0)),
            pl.BlockSpec((_BB, _L1), lambda i: (i + _BATCH // _BB, 0)),
            pl.BlockSpec((32, _L1), lambda i: (0, 0)),
            pl.BlockSpec((32, _L1), lambda i: (0, 0)),
            pl.BlockSpec((1, 32), lambda i: (0, 0)),
            pl.BlockSpec((32, 32), lambda i: (0, 0)),
            pl.BlockSpec((1, 32), lambda i: (0, 0)),
            pl.BlockSpec((1, 32), lambda i: (0, 0)),
            pl.BlockSpec((1, 1), lambda i: (0, 0)),
        ],
        out_specs=pl.BlockSpec((_BB, 1), lambda i: (i, 0)),
        out_shape=jax.ShapeDtypeStruct((_BATCH, 1), jnp.float32),
    )(y, y, w1s, w1n, b1, w2, b2, wo, bo)


def kernel(stm_indices, nstm_indices, emb, feature_bias,
           l1_w, l1_b, l2_w, l2_b, out_w, out_b):
    idx = jnp.concatenate([stm_indices, nstm_indices], axis=0)
    idx = idx.reshape(-1).astype(jnp.int32)
    y = _sc_bag_sum(idx, feature_bias, emb)
    w1s = l1_w[:, :_L1]
    w1n = l1_w[:, _L1:]
    return _mlp(y, w1s, w1n, l1_b.reshape(1, 32), l2_w, l2_b.reshape(1, 32),
                out_w.reshape(1, 32), out_b.reshape(1, 1))


# speedup vs baseline: 8.5463x; 8.5463x over previous
import functools

import jax
import jax.numpy as jnp
from jax import lax
from jax.experimental import pallas as pl
from jax.experimental.pallas import tpu as pltpu
from jax.experimental.pallas import tpu_sc as plsc

_INPUT_SIZE = 41024
_L1 = 256
_BATCH = 16384
_N_ACTIVE = 32

_NC = 2
_NS = 16
_NW = _NC * _NS

_TOTAL_BAGS = 2 * _BATCH
_BAGS_PER_W = _TOTAL_BAGS // _NW
_C = 4
_NCHUNK = _BAGS_PER_W // _C
_ROWS_PER_CHUNK = _C * _N_ACTIVE


def _sc_bag_sum_body(idx_hbm, bias_hbm, emb_hbm, out_hbm,
                     idx_v, rows_v, acc_v, bias_v, sem0, sem1):
    wid = lax.axis_index("s") * _NC + lax.axis_index("c")
    base_bag = wid * _BAGS_PER_W

    pltpu.sync_copy(idx_hbm.at[pl.ds(base_bag * _N_ACTIVE, _BAGS_PER_W * _N_ACTIVE)],
                    idx_v)
    pltpu.sync_copy(bias_hbm, bias_v)

    sems = (sem0, sem1)

    def issue_gather(c, b):
        off = c * _ROWS_PER_CHUNK
        pltpu.async_copy(emb_hbm.at[idx_v.at[pl.ds(off, _ROWS_PER_CHUNK)]],
                         rows_v.at[b], sems[b])

    issue_gather(0, 0)
    issue_gather(1, 1)

    def outer(i, carry):
        for b in range(2):
            c = 2 * i + b
            pltpu.make_async_copy(emb_hbm.at[pl.ds(0, _ROWS_PER_CHUNK)],
                                  rows_v.at[b], sems[b]).wait()
            for j in range(_C):
                def row_add(r, acc, _j=j, _b=b):
                    return tuple(
                        acc[k] + rows_v[_b, _j * _N_ACTIVE + r, pl.ds(k * 16, 16)]
                        for k in range(_L1 // 16))
                acc0 = tuple(bias_v[pl.ds(k * 16, 16)] for k in range(_L1 // 16))
                acc = lax.fori_loop(0, _N_ACTIVE, row_add, acc0)
                for k in range(_L1 // 16):
                    v = jnp.minimum(jnp.maximum(acc[k], 0.0), 1.0)
                    acc_v[j, pl.ds(k * 16, 16)] = v
            pltpu.sync_copy(acc_v, out_hbm.at[pl.ds(base_bag + c * _C, _C)])
            @pl.when(c + 2 < _NCHUNK)
            def _():
                issue_gather(c + 2, b)
        return carry

    lax.fori_loop(0, _NCHUNK // 2, outer, 0)


@functools.lru_cache(maxsize=None)
def _sc_bag_sum_fn():
    return pl.kernel(
        _sc_bag_sum_body,
        out_type=jax.ShapeDtypeStruct((_TOTAL_BAGS, _L1), jnp.float32),
        mesh=plsc.VectorSubcoreMesh(core_axis_name="c", subcore_axis_name="s",
                                    num_cores=_NC, num_subcores=_NS),
        scratch_types=[
            pltpu.VMEM((_BAGS_PER_W * _N_ACTIVE,), jnp.int32),
            pltpu.VMEM((2, _ROWS_PER_CHUNK, _L1), jnp.float32),
            pltpu.VMEM((_C, _L1), jnp.float32),
            pltpu.VMEM((_L1,), jnp.float32),
            pltpu.SemaphoreType.DMA,
            pltpu.SemaphoreType.DMA,
        ],
    )


def _mlp_body(ys_ref, yn_ref, w1s_ref, w1n_ref, b1_ref, w2_ref, b2_ref,
              wo_ref, bo_ref, out_ref):
    dn = (((1,), (1,)), ((), ()))
    h = (lax.dot_general(ys_ref[...], w1s_ref[...], dn,
                         preferred_element_type=jnp.float32)
         + lax.dot_general(yn_ref[...], w1n_ref[...], dn,
                           preferred_element_type=jnp.float32)
         + b1_ref[...])
    h = jnp.clip(h, 0.0, 1.0)
    h = lax.dot_general(h, w2_ref[...], dn,
                        preferred_element_type=jnp.float32) + b2_ref[...]
    h = jnp.clip(h, 0.0, 1.0)
    o = lax.dot_general(h, wo_ref[...], (((1,), (0,)), ((), ())),
                        preferred_element_type=jnp.float32) + bo_ref[0, 0]
    out_ref[...] = jax.nn.sigmoid(o)


_BB = 2048


def _mlp(y, w1s, w1n, b1, w2, b2, wo, bo):
    grid = (_BATCH // _BB,)
    return pl.pallas_call(
        _mlp_body,
        grid=grid,
        in_specs=[
            pl.BlockSpec((_BB, _L1), lambda i: (i, 0)),
            pl.BlockSpec((_BB, _L1), lambda i: (i + _BATCH // _BB, 0)),
            pl.BlockSpec((32, _L1), lambda i: (0, 0)),
            pl.BlockSpec((32, _L1), lambda i: (0, 0)),
            pl.BlockSpec((1, 32), lambda i: (0, 0)),
            pl.BlockSpec((32, 32), lambda i: (0, 0)),
            pl.BlockSpec((1, 32), lambda i: (0, 0)),
            pl.BlockSpec((32, 1), lambda i: (0, 0)),
            pl.BlockSpec((1, 1), lambda i: (0, 0)),
        ],
        out_specs=pl.BlockSpec((_BB, 1), lambda i: (i, 0)),
        out_shape=jax.ShapeDtypeStruct((_BATCH, 1), jnp.float32),
    )(y, y, w1s, w1n, b1, w2, b2, wo, bo)


def kernel(stm_indices, nstm_indices, emb, feature_bias,
           l1_w, l1_b, l2_w, l2_b, out_w, out_b):
    idx = jnp.concatenate([stm_indices, nstm_indices], axis=0)
    idx = idx.reshape(-1).astype(jnp.int32)
    y = _sc_bag_sum_fn()(idx, feature_bias, emb)
    w1s = l1_w[:, :_L1]
    w1n = l1_w[:, _L1:]
    return _mlp(y, w1s, w1n, l1_b.reshape(1, 32), l2_w, l2_b.reshape(1, 32),
                out_w.reshape(1, 32).T, out_b.reshape(1, 1))
